# baseline (speedup 1.0000x reference)
import functools
import jax
import jax.numpy as jnp
from jax import lax
from jax.experimental import pallas as pl
from jax.experimental.pallas import tpu as pltpu
from jax.experimental.pallas import tpu_sc as plsc

BATCH = 4
SEQ_LEN = 8192
D_MODEL = 768

N_CORES = 2
N_SUBCORES = 16
N_WORKERS = N_CORES * N_SUBCORES
ROWS_PER_W = SEQ_LEN // N_WORKERS
CHUNK_ROWS = 32
N_CHUNKS = ROWS_PER_W // CHUNK_ROWS
CHUNK_W = CHUNK_ROWS * D_MODEL
N_VECS = CHUNK_W // 16
D_X = 3
NITER = N_CHUNKS * BATCH
XSTRIDE = SEQ_LEN * D_MODEL


def _sc_body(x_ref, pe_ref, out_ref, xbufs, pebufs, xsems, osems, psems):
    wid = lax.axis_index("s") * N_CORES + lax.axis_index("c")
    base = wid * ROWS_PER_W * D_MODEL

    def x_in(k):
        c, b = divmod(k, BATCH)
        return pltpu.make_async_copy(
            x_ref.at[pl.ds(b * XSTRIDE + base + c * CHUNK_W, CHUNK_W)],
            xbufs[k % D_X],
            xsems[k % D_X],
        )

    def x_out(k):
        c, b = divmod(k, BATCH)
        return pltpu.make_async_copy(
            xbufs[k % D_X],
            out_ref.at[pl.ds(b * XSTRIDE + base + c * CHUNK_W, CHUNK_W)],
            osems[k % D_X],
        )

    def pe_in(c):
        return pltpu.make_async_copy(
            pe_ref.at[pl.ds(base + c * CHUNK_W, CHUNK_W)],
            pebufs[c % 2],
            psems[c % 2],
        )

    for k in range(D_X):
        x_in(k).start()
    pe_in(0).start()
    pe_in(1).start()

    for k in range(NITER):
        c, b = divmod(k, BATCH)

        m = k - 1
        if m >= 0 and m + D_X < NITER:
            x_out(m).wait()
            x_in(m + D_X).start()

        x_in(k).wait()
        if b == 0:
            pe_in(c).wait()

        xb, pb = xbufs[k % D_X], pebufs[c % 2]

        def add8(i, _):
            for j in range(8):
                sl = pl.ds((i * 8 + j) * 16, 16)
                plsc.addupdate(xb.at[sl], pb[sl])
            return 0

        lax.fori_loop(0, N_VECS // 8, add8, 0)

        x_out(k).start()
        if b == BATCH - 1 and c + 2 < N_CHUNKS:
            pe_in(c + 2).start()

    for k in range(max(NITER - D_X, 0), NITER):
        x_out(k).wait()


@functools.partial(
    pl.kernel,
    out_type=jax.ShapeDtypeStruct((BATCH * SEQ_LEN * D_MODEL,), jnp.float32),
    mesh=plsc.VectorSubcoreMesh(core_axis_name="c", subcore_axis_name="s"),
    scratch_types=[
        [pltpu.VMEM((CHUNK_W,), jnp.float32)] * D_X,
        [pltpu.VMEM((CHUNK_W,), jnp.float32)] * 2,
        [pltpu.SemaphoreType.DMA] * D_X,
        [pltpu.SemaphoreType.DMA] * D_X,
        [pltpu.SemaphoreType.DMA] * 2,
    ],
)
def _sc_add(x_ref, pe_ref, out_ref, xbufs, pebufs, xsems, osems, psems):
    _sc_body(x_ref, pe_ref, out_ref, xbufs, pebufs, xsems, osems, psems)


def kernel(x, pe):
    out = _sc_add(x.reshape(-1), pe.reshape(-1))
    return out.reshape(BATCH, SEQ_LEN, D_MODEL)


# speedup vs baseline: 4.6998x; 4.6998x over previous
import jax
import jax.numpy as jnp
from jax.experimental import pallas as pl
from jax.experimental.pallas import tpu as pltpu

BATCH = 4
SEQ_LEN = 8192
D_MODEL = 768
CH_S = 1024
N_SC = SEQ_LEN // CH_S
NITER = N_SC * BATCH
D_IN = 8
D_OUT = 8


def _body(x_hbm, pe_hbm, o_hbm, xbufs, obufs, pebufs, insems, outsems, pesems):
    def in_copy(k):
        s, b = divmod(k, BATCH)
        slot = k % D_IN
        return pltpu.make_async_copy(
            x_hbm.at[b, pl.ds(s * CH_S, CH_S)], xbufs.at[slot], insems.at[slot]
        )

    def out_copy(k):
        s, b = divmod(k, BATCH)
        slot = k % D_OUT
        return pltpu.make_async_copy(
            obufs.at[slot], o_hbm.at[b, pl.ds(s * CH_S, CH_S)], outsems.at[slot]
        )

    def pe_copy(s):
        return pltpu.make_async_copy(
            pe_hbm.at[pl.ds(s * CH_S, CH_S)], pebufs.at[s % 2], pesems.at[s % 2]
        )

    for k in range(D_IN):
        in_copy(k).start()
    pe_copy(0).start()
    pe_copy(1).start()

    for k in range(NITER):
        s, b = divmod(k, BATCH)
        islot, oslot = k % D_IN, k % D_OUT

        in_copy(k).wait()
        if b == 0:
            pe_copy(s).wait()
        if k >= D_OUT:
            out_copy(k - D_OUT).wait()

        obufs[oslot] = xbufs[islot] + pebufs[s % 2]
        out_copy(k).start()

        if k + D_IN < NITER:
            in_copy(k + D_IN).start()
        if b == BATCH - 1 and s + 2 < N_SC:
            pe_copy(s + 2).start()

    for k in range(max(NITER - D_OUT, 0), NITER):
        out_copy(k).wait()


def kernel(x, pe):
    return pl.pallas_call(
        _body,
        in_specs=[
            pl.BlockSpec(memory_space=pl.ANY),
            pl.BlockSpec(memory_space=pl.ANY),
        ],
        out_specs=pl.BlockSpec(memory_space=pl.ANY),
        out_shape=jax.ShapeDtypeStruct((BATCH, SEQ_LEN, D_MODEL), x.dtype),
        scratch_shapes=[
            pltpu.VMEM((D_IN, CH_S, D_MODEL), jnp.float32),
            pltpu.VMEM((D_OUT, CH_S, D_MODEL), jnp.float32),
            pltpu.VMEM((2, CH_S, D_MODEL), jnp.float32),
            pltpu.SemaphoreType.DMA((D_IN,)),
            pltpu.SemaphoreType.DMA((D_OUT,)),
            pltpu.SemaphoreType.DMA((2,)),
        ],
    )(x, pe)


# speedup vs baseline: 4.7805x; 1.0172x over previous
import jax
import jax.numpy as jnp
from jax.experimental import pallas as pl
from jax.experimental.pallas import tpu as pltpu

BATCH = 4
SEQ_LEN = 8192
D_MODEL = 768
CH_S = 1024
N_SC = SEQ_LEN // CH_S
NITER = N_SC * BATCH
D_IN = 6
D_OUT = 6


def _body(x_hbm, pe_hbm, o_hbm, xbufs, obufs, pebufs, insems, outsems, pesems):
    def in_copy(k):
        s, b = divmod(k, BATCH)
        slot = k % D_IN
        return pltpu.make_async_copy(
            x_hbm.at[b, pl.ds(s * CH_S, CH_S)], xbufs.at[slot], insems.at[slot]
        )

    def out_copy(k):
        s, b = divmod(k, BATCH)
        slot = k % D_OUT
        return pltpu.make_async_copy(
            obufs.at[slot], o_hbm.at[b, pl.ds(s * CH_S, CH_S)], outsems.at[slot]
        )

    def pe_copy(s):
        return pltpu.make_async_copy(
            pe_hbm.at[pl.ds(s * CH_S, CH_S)], pebufs.at[s % 2], pesems.at[s % 2]
        )

    for k in range(D_IN):
        in_copy(k).start()
    pe_copy(0).start()
    pe_copy(1).start()

    for k in range(NITER):
        s, b = divmod(k, BATCH)
        islot, oslot = k % D_IN, k % D_OUT

        in_copy(k).wait()
        if b == 0:
            pe_copy(s).wait()
        if k >= D_OUT:
            out_copy(k - D_OUT).wait()

        obufs[oslot] = xbufs[islot] + pebufs[s % 2]
        out_copy(k).start()

        if k + D_IN < NITER:
            in_copy(k + D_IN).start()
        if b == BATCH - 1 and s + 2 < N_SC:
            pe_copy(s + 2).start()

    for k in range(max(NITER - D_OUT, 0), NITER):
        out_copy(k).wait()


def kernel(x, pe):
    return pl.pallas_call(
        _body,
        in_specs=[
            pl.BlockSpec(memory_space=pl.ANY),
            pl.BlockSpec(memory_space=pl.ANY),
        ],
        out_specs=pl.BlockSpec(memory_space=pl.ANY),
        out_shape=jax.ShapeDtypeStruct((BATCH, SEQ_LEN, D_MODEL), x.dtype),
        scratch_shapes=[
            pltpu.VMEM((D_IN, CH_S, D_MODEL), jnp.float32),
            pltpu.VMEM((D_OUT, CH_S, D_MODEL), jnp.float32),
            pltpu.VMEM((2, CH_S, D_MODEL), jnp.float32),
            pltpu.SemaphoreType.DMA((D_IN,)),
            pltpu.SemaphoreType.DMA((D_OUT,)),
            pltpu.SemaphoreType.DMA((2,)),
        ],
    )(x, pe)
